# Initial kernel scaffold; baseline (speedup 1.0000x reference)
import functools

import jax
import jax.numpy as jnp
from jax import lax
from jax.experimental import pallas as pl
from jax.experimental.pallas import tpu as pltpu
from jax.experimental.pallas import tpu_sc as plsc

NC, NS = 2, 16
NW = NC * NS


def kernel(indices, weights, table):
    B, L = indices.shape
    V, D = table.shape
    assert D == 32 and B % NW == 0
    SPW = B // NW
    CH = 32
    assert SPW % CH == 0
    NCHUNK = SPW // CH
    IPC = CH * L
    GSUB = 80
    assert IPC % GSUB == 0 and GSUB % 8 == 0
    NG = IPC // GSUB

    idx_flat = indices.reshape(B * L)
    w_flat = weights.reshape(B * L)

    mesh = plsc.VectorSubcoreMesh(core_axis_name="c", subcore_axis_name="s")

    @functools.partial(
        pl.kernel,
        mesh=mesh,
        out_type=jax.ShapeDtypeStruct((B, D), jnp.float32),
        scratch_types=[
            pltpu.VMEM((IPC,), jnp.int32),
            pltpu.VMEM((IPC,), jnp.float32),
            pltpu.VMEM((IPC, D), jnp.float32),
            pltpu.VMEM((CH, D), jnp.float32),
            pltpu.SemaphoreType.DMA,
        ],
    )
    def sc_kernel(idx_hbm, w_hbm, table_hbm, out_hbm, idx_v, w_v, rows_v,
                  out_v, gsem):
        wid = lax.axis_index("s") * NC + lax.axis_index("c")
        base = wid * SPW * L

        def chunk_body(ci, carry):
            off = base + ci * IPC
            pltpu.sync_copy(idx_hbm.at[pl.ds(off, IPC)], idx_v)
            pltpu.sync_copy(w_hbm.at[pl.ds(off, IPC)], w_v)
            copies = []
            for g in range(NG):
                copies.append(pltpu.async_copy(
                    table_hbm.at[idx_v.at[pl.ds(g * GSUB, GSUB)]],
                    rows_v.at[pl.ds(g * GSUB, GSUB)],
                    gsem,
                ))
            for c in copies:
                c.wait()

            def sample_body(s, carry2):
                jbase = s * L

                def l_body(l, accs):
                    a0, a1 = accs
                    j = jbase + l
                    w = jnp.broadcast_to(w_v[j], (16,))
                    r0 = rows_v[j, pl.ds(0, 16)]
                    r1 = rows_v[j, pl.ds(16, 16)]
                    return (a0 + w * r0, a1 + w * r1)

                a0, a1 = lax.fori_loop(
                    0, L, l_body,
                    (jnp.zeros((16,), jnp.float32),
                     jnp.zeros((16,), jnp.float32)))
                out_v[s, pl.ds(0, 16)] = a0
                out_v[s, pl.ds(16, 16)] = a1
                return carry2

            lax.fori_loop(0, CH, sample_body, 0)
            pltpu.sync_copy(out_v, out_hbm.at[pl.ds(wid * SPW + ci * CH, CH)])
            return carry

        lax.fori_loop(0, NCHUNK, chunk_body, 0)

    return sc_kernel(idx_flat, w_flat, table)


# speedup vs baseline: 2.6957x; 2.6957x over previous
import functools

import jax
import jax.numpy as jnp
from jax import lax
from jax.experimental import pallas as pl
from jax.experimental.pallas import tpu as pltpu
from jax.experimental.pallas import tpu_sc as plsc

NC, NS = 2, 16
NW = NC * NS


def kernel(indices, weights, table):
    B, L = indices.shape
    V, D = table.shape
    assert D == 32 and B % NW == 0
    SPW = B // NW
    CH = 32
    assert SPW % CH == 0
    NCHUNK = SPW // CH
    IPC = CH * L
    GSUB = 80
    assert IPC % GSUB == 0 and GSUB % 8 == 0
    NG = IPC // GSUB

    idx_flat = indices.reshape(B * L)
    w_flat = weights.reshape(B * L)

    mesh = plsc.VectorSubcoreMesh(core_axis_name="c", subcore_axis_name="s")

    @functools.partial(
        pl.kernel,
        mesh=mesh,
        out_type=jax.ShapeDtypeStruct((B, D), jnp.float32),
        compiler_params=pltpu.CompilerParams(use_tc_tiling_on_sc=False),
        scratch_types=[
            pltpu.VMEM((IPC,), jnp.int32),
            pltpu.VMEM((IPC + 16,), jnp.float32),
            pltpu.VMEM((IPC, D), jnp.float32),
            pltpu.VMEM((CH, D), jnp.float32),
            pltpu.SemaphoreType.DMA,
        ],
    )
    def sc_kernel(idx_hbm, w_hbm, table_hbm, out_hbm, idx_v, w_v, rows_v,
                  out_v, gsem):
        wid = lax.axis_index("s") * NC + lax.axis_index("c")
        base = wid * SPW * L

        def chunk_body(ci, carry):
            off = base + ci * IPC
            pltpu.sync_copy(idx_hbm.at[pl.ds(off, IPC)], idx_v)
            pltpu.sync_copy(w_hbm.at[pl.ds(off, IPC)], w_v.at[pl.ds(0, IPC)])
            copies = []
            for g in range(NG):
                copies.append(pltpu.async_copy(
                    table_hbm.at[idx_v.at[pl.ds(g * GSUB, GSUB)]],
                    rows_v.at[pl.ds(g * GSUB, GSUB)],
                    gsem,
                ))
            for c in copies:
                c.wait()

            def sample_body(s, carry2):
                jbase = s * L
                a0 = jnp.zeros((16,), jnp.float32)
                a1 = jnp.zeros((16,), jnp.float32)
                for lg in range((L + 15) // 16):
                    cnt = min(16, L - lg * 16)
                    wvec = w_v[pl.ds(jbase + lg * 16, 16)]
                    for i in range(cnt):
                        w = jnp.broadcast_to(wvec[i], (16,))
                        j = jbase + lg * 16 + i
                        r0 = rows_v[j, pl.ds(0, 16)]
                        r1 = rows_v[j, pl.ds(16, 16)]
                        a0 = a0 + w * r0
                        a1 = a1 + w * r1
                out_v[s, pl.ds(0, 16)] = a0
                out_v[s, pl.ds(16, 16)] = a1
                return carry2

            lax.fori_loop(0, CH, sample_body, 0)
            pltpu.sync_copy(out_v, out_hbm.at[pl.ds(wid * SPW + ci * CH, CH)])
            return carry

        lax.fori_loop(0, NCHUNK, chunk_body, 0)

    return sc_kernel(idx_flat, w_flat, table)
